# Initial kernel scaffold; baseline (speedup 1.0000x reference)
import functools

import jax
import jax.numpy as jnp
from jax import lax
from jax.experimental import pallas as pl
from jax.experimental.pallas import tpu as pltpu
from jax.experimental.pallas import tpu_sc as plsc

N = 10000
E = 320000
DI = 128
DH = 128
DO = 64

NC, NS = 2, 16
NW = NC * NS
NP = 10240
RPT = NP // NS
CH = 128
NSTEP = 80
NBUF = 4
EPW = NSTEP * CH
EPAD = NW * EPW
TRASH = NP - 1

RB = 1024
GRID = NP // RB


def _mesh():
    return plsc.VectorSubcoreMesh(
        core_axis_name="c", subcore_axis_name="s", num_cores=NC, num_subcores=NS
    )


def _deg_body(dst_hbm, zeros_hbm, ones_hbm, out_hbm, idx_v, ones_v, deg_sh, sem):
    cid = lax.axis_index("c")
    sid = lax.axis_index("s")
    wid = cid * NS + sid
    pltpu.sync_copy(dst_hbm.at[wid], idx_v)
    pltpu.sync_copy(ones_hbm, ones_v)
    pltpu.sync_copy(zeros_hbm, deg_sh.at[pl.ds(sid * RPT, RPT)])
    plsc.subcore_barrier()

    def fire(j, carry):
        pltpu.async_copy(ones_v, deg_sh.at[idx_v.at[j]], sem, add=True)
        return carry

    lax.fori_loop(0, NSTEP, fire, 0)

    def drain(j, carry):
        pltpu.make_async_copy(ones_v, deg_sh.at[idx_v.at[0]], sem).wait()
        return carry

    lax.fori_loop(0, NSTEP, drain, 0)
    plsc.subcore_barrier()
    pltpu.sync_copy(
        deg_sh.at[pl.ds(sid * RPT, RPT)], out_hbm.at[cid, pl.ds(sid * RPT, RPT)]
    )


def _deg_kernel(dst3, za, ones):
    return pl.kernel(
        _deg_body,
        out_type=jax.ShapeDtypeStruct((NC, NP), jnp.float32),
        mesh=_mesh(),
        scratch_types=[
            pltpu.VMEM((NSTEP, CH), jnp.int32),
            pltpu.VMEM((CH,), jnp.float32),
            pltpu.VMEM_SHARED((NP,), jnp.float32),
            pltpu.SemaphoreType.DMA,
        ],
    )(dst3, za, ones)


def _scat_body(xt_hbm, src_hbm, dst_hbm, zeros_hbm, out_hbm,
               sidx, didx, r0, r1, r2, r3,
               acc_sh, sg0, sg1, sg2, sg3, ss0, ss1, ss2, ss3):
    rows = (r0, r1, r2, r3)
    semg = (sg0, sg1, sg2, sg3)
    sems = (ss0, ss1, ss2, ss3)
    cid = lax.axis_index("c")
    sid = lax.axis_index("s")
    wid = cid * NS + sid
    pltpu.sync_copy(src_hbm.at[wid], sidx)
    pltpu.sync_copy(dst_hbm.at[wid], didx)
    pltpu.sync_copy(zeros_hbm, acc_sh.at[pl.ds(sid * RPT, RPT)])
    plsc.subcore_barrier()

    for b in range(NBUF):
        pltpu.async_copy(xt_hbm.at[sidx.at[b]], rows[b], semg[b])

    def body(j, carry):
        k = j * NBUF
        for b in range(NBUF):
            pltpu.make_async_copy(xt_hbm.at[sidx.at[0]], rows[b], semg[b]).wait()
            pltpu.async_copy(rows[b], acc_sh.at[didx.at[k + b]], sems[b], add=True)
        for b in range(NBUF):
            nxt = jnp.minimum(k + NBUF + b, NSTEP - 1)
            pltpu.make_async_copy(rows[b], acc_sh.at[didx.at[0]], sems[b]).wait()
            pltpu.async_copy(xt_hbm.at[sidx.at[nxt]], rows[b], semg[b])
        return carry

    lax.fori_loop(0, NSTEP // NBUF, body, 0)
    for b in range(NBUF):
        pltpu.make_async_copy(xt_hbm.at[sidx.at[0]], rows[b], semg[b]).wait()
    plsc.subcore_barrier()
    pltpu.sync_copy(
        acc_sh.at[pl.ds(sid * RPT, RPT)], out_hbm.at[cid, pl.ds(sid * RPT, RPT)]
    )


def _scat_kernel(xt, src3, dst3, zc):
    return pl.kernel(
        _scat_body,
        out_type=jax.ShapeDtypeStruct((NC, NP, DH), jnp.float32),
        mesh=_mesh(),
        scratch_types=[
            pltpu.VMEM((NSTEP, CH), jnp.int32),
            pltpu.VMEM((NSTEP, CH), jnp.int32),
            pltpu.VMEM((CH, DH), jnp.float32),
            pltpu.VMEM((CH, DH), jnp.float32),
            pltpu.VMEM((CH, DH), jnp.float32),
            pltpu.VMEM((CH, DH), jnp.float32),
            pltpu.VMEM_SHARED((NP, DH), jnp.float32),
            pltpu.SemaphoreType.DMA,
            pltpu.SemaphoreType.DMA,
            pltpu.SemaphoreType.DMA,
            pltpu.SemaphoreType.DMA,
            pltpu.SemaphoreType.DMA,
            pltpu.SemaphoreType.DMA,
            pltpu.SemaphoreType.DMA,
            pltpu.SemaphoreType.DMA,
        ],
    )(xt, src3, dst3, zc)


def _mm1_body(h_ref, w_ref, da_ref, db_ref, o_ref):
    dis = lax.rsqrt(da_ref[...] + db_ref[...] + 1.0)
    o_ref[...] = (
        jnp.dot(h_ref[...], w_ref[...], preferred_element_type=jnp.float32) * dis
    )


def _mm1(hp, W, dA, dB):
    return pl.pallas_call(
        _mm1_body,
        grid=(GRID,),
        in_specs=[
            pl.BlockSpec((RB, DI), lambda i: (i, 0)),
            pl.BlockSpec((DI, DH), lambda i: (0, 0)),
            pl.BlockSpec((RB, 1), lambda i: (i, 0)),
            pl.BlockSpec((RB, 1), lambda i: (i, 0)),
        ],
        out_specs=pl.BlockSpec((RB, DH), lambda i: (i, 0)),
        out_shape=jax.ShapeDtypeStruct((NP, DH), jnp.float32),
    )(hp, W, dA, dB)


def _mm2_body(acc_ref, xt_ref, da_ref, db_ref, bg_ref, wf_ref, bf_ref, o_ref):
    dis = lax.rsqrt(da_ref[...] + db_ref[...] + 1.0)
    a = (acc_ref[0] + acc_ref[1] + xt_ref[...]) * dis + bg_ref[...]
    z = jnp.maximum(a, 0.0)
    o_ref[...] = (
        jnp.dot(z, wf_ref[...], preferred_element_type=jnp.float32) + bf_ref[...]
    )


def _mm2(acc, xt, dA, dB, bg, Wf, bf):
    return pl.pallas_call(
        _mm2_body,
        grid=(GRID,),
        in_specs=[
            pl.BlockSpec((NC, RB, DH), lambda i: (0, i, 0)),
            pl.BlockSpec((RB, DH), lambda i: (i, 0)),
            pl.BlockSpec((RB, 1), lambda i: (i, 0)),
            pl.BlockSpec((RB, 1), lambda i: (i, 0)),
            pl.BlockSpec((1, DH), lambda i: (0, 0)),
            pl.BlockSpec((DH, DO), lambda i: (0, 0)),
            pl.BlockSpec((1, DO), lambda i: (0, 0)),
        ],
        out_specs=pl.BlockSpec((RB, DO), lambda i: (i, 0)),
        out_shape=jax.ShapeDtypeStruct((NP, DO), jnp.float32),
    )(acc, xt, dA, dB, bg, Wf, bf)


def kernel(h, edge_index, W_gcn, b_gcn, W_fc, b_fc):
    h = h.astype(jnp.float32)
    ei = edge_index.astype(jnp.int32)
    src3 = jnp.pad(ei[0], (0, EPAD - E)).reshape(NW, NSTEP, CH)
    dst3 = jnp.pad(ei[1], (0, EPAD - E), constant_values=TRASH).reshape(
        NW, NSTEP, CH
    )
    hp = jnp.pad(h, ((0, NP - N), (0, 0)))
    za = jnp.zeros((RPT,), jnp.float32)
    zc = jnp.zeros((RPT, DH), jnp.float32)
    ones = jnp.ones((CH,), jnp.float32)

    deg = _deg_kernel(dst3, za, ones)
    dA = deg[0].reshape(NP, 1)
    dB = deg[1].reshape(NP, 1)
    xt = _mm1(hp, W_gcn, dA, dB)
    acc = _scat_kernel(xt, src3, dst3, zc)
    out = _mm2(acc, xt, dA, dB, b_gcn.reshape(1, DH), W_fc, b_fc.reshape(1, DO))
    return out[:N]


# speedup vs baseline: 3.4731x; 3.4731x over previous
import jax
import jax.numpy as jnp
from jax import lax
from jax.experimental import pallas as pl
from jax.experimental.pallas import tpu as pltpu
from jax.experimental.pallas import tpu_sc as plsc

N = 10000
E = 320000
DI = 128
DH = 128
DO = 64

NC, NS = 2, 16
NW = NC * NS
NP = 10240
RPT = NP // NS
CH = 128
NSTEP_D = 80
NSTEP = 160
NBUF = 4
EPAD = NW * NSTEP_D * CH
TRASH = NP - 1

RB = 1024
GRID = NP // RB


def _deg_body(dst_hbm, zeros_hbm, ones_hbm, out_hbm, idx_v, ones_v, deg_sh, sem):
    cid = lax.axis_index("c")
    sid = lax.axis_index("s")
    wid = cid * NS + sid
    pltpu.sync_copy(dst_hbm.at[wid], idx_v)
    pltpu.sync_copy(ones_hbm, ones_v)
    pltpu.sync_copy(zeros_hbm, deg_sh.at[pl.ds(sid * RPT, RPT)])
    plsc.subcore_barrier()

    def fire(j, carry):
        pltpu.async_copy(ones_v, deg_sh.at[idx_v.at[j]], sem, add=True)
        return carry

    lax.fori_loop(0, NSTEP_D, fire, 0)

    def drain(j, carry):
        pltpu.make_async_copy(ones_v, deg_sh.at[idx_v.at[0]], sem).wait()
        return carry

    lax.fori_loop(0, NSTEP_D, drain, 0)
    plsc.subcore_barrier()
    pltpu.sync_copy(
        deg_sh.at[pl.ds(sid * RPT, RPT)], out_hbm.at[cid, pl.ds(sid * RPT, RPT)]
    )


def _deg_kernel(dst3d, za, ones):
    return pl.kernel(
        _deg_body,
        out_type=jax.ShapeDtypeStruct((NC, NP), jnp.float32),
        mesh=plsc.VectorSubcoreMesh(
            core_axis_name="c", subcore_axis_name="s", num_cores=NC, num_subcores=NS
        ),
        scratch_types=[
            pltpu.VMEM((NSTEP_D, CH), jnp.int32),
            pltpu.VMEM((CH,), jnp.float32),
            pltpu.VMEM_SHARED((NP,), jnp.float32),
            pltpu.SemaphoreType.DMA,
        ],
    )(dst3d, za, ones)


PASS_ROWS = 2560
NPASS = NP // PASS_ROWS
TR = 128
ACCR = PASS_ROWS + TR
ZR = ACCR // NS
CPT = PASS_ROWS // NS


def _scat_body(xt_hbm, src_hbm, dst_hbm, zeros_hbm, out_hbm,
               sidx, didx, r0, r1, r2, r3,
               acc_sh, sg0, sg1, sg2, sg3, ss0, ss1, ss2, ss3):
    rows = (r0, r1, r2, r3)
    semg = (sg0, sg1, sg2, sg3)
    sems = (ss0, ss1, ss2, ss3)
    sid = lax.axis_index("s")
    pltpu.sync_copy(src_hbm.at[sid], sidx)

    def one_pass(p, carry):
        base = p * PASS_ROWS
        pltpu.sync_copy(dst_hbm.at[sid], didx)
        pltpu.sync_copy(zeros_hbm, acc_sh.at[pl.ds(sid * ZR, CH)])
        pltpu.sync_copy(
            zeros_hbm.at[pl.ds(0, ZR - CH)],
            acc_sh.at[pl.ds(sid * ZR + CH, ZR - CH)],
        )

        def remap(r, c):
            for q in range(CH // 16):
                d = didx[r, pl.ds(q * 16, 16)]
                l = d - base
                ok = jnp.logical_and(l >= 0, l < PASS_ROWS)
                t = PASS_ROWS + jnp.bitwise_and(d, TR - 1)
                didx[r, pl.ds(q * 16, 16)] = jnp.where(ok, l, t)
            return c

        lax.fori_loop(0, NSTEP, remap, 0)
        plsc.subcore_barrier()

        for b in range(NBUF):
            pltpu.async_copy(xt_hbm.at[sidx.at[b]], rows[b], semg[b])

        def body(j, carry2):
            k = j * NBUF
            for b in range(NBUF):
                pltpu.make_async_copy(
                    xt_hbm.at[sidx.at[0]], rows[b], semg[b]
                ).wait()
                pltpu.async_copy(
                    rows[b], acc_sh.at[didx.at[k + b]], sems[b], add=True
                )
            for b in range(NBUF):
                nxt = jnp.minimum(k + NBUF + b, NSTEP - 1)
                pltpu.make_async_copy(
                    rows[b], acc_sh.at[didx.at[0]], sems[b]
                ).wait()
                pltpu.async_copy(xt_hbm.at[sidx.at[nxt]], rows[b], semg[b])
            return carry2

        lax.fori_loop(0, NSTEP // NBUF, body, 0)
        for b in range(NBUF):
            pltpu.make_async_copy(xt_hbm.at[sidx.at[0]], rows[b], semg[b]).wait()
        plsc.subcore_barrier()

        pltpu.sync_copy(
            acc_sh.at[pl.ds(sid * CPT, CPT)],
            out_hbm.at[pl.ds(base + sid * CPT, CPT)],
        )
        plsc.subcore_barrier()
        return carry

    lax.fori_loop(0, NPASS, one_pass, 0)


def _scat_kernel(xt, src3, dst3, zb):
    return pl.kernel(
        _scat_body,
        out_type=jax.ShapeDtypeStruct((NP, DH), jnp.float32),
        mesh=plsc.VectorSubcoreMesh(
            core_axis_name="c", subcore_axis_name="s", num_cores=1, num_subcores=NS
        ),
        scratch_types=[
            pltpu.VMEM((NSTEP, CH), jnp.int32),
            pltpu.VMEM((NSTEP, CH), jnp.int32),
            pltpu.VMEM((CH, DH), jnp.float32),
            pltpu.VMEM((CH, DH), jnp.float32),
            pltpu.VMEM((CH, DH), jnp.float32),
            pltpu.VMEM((CH, DH), jnp.float32),
            pltpu.VMEM_SHARED((ACCR, DH), jnp.float32),
            pltpu.SemaphoreType.DMA,
            pltpu.SemaphoreType.DMA,
            pltpu.SemaphoreType.DMA,
            pltpu.SemaphoreType.DMA,
            pltpu.SemaphoreType.DMA,
            pltpu.SemaphoreType.DMA,
            pltpu.SemaphoreType.DMA,
            pltpu.SemaphoreType.DMA,
        ],
    )(xt, src3, dst3, zb)


def _mm1_body(h_ref, w_ref, da_ref, db_ref, o_ref):
    dis = lax.rsqrt(da_ref[...] + db_ref[...] + 1.0)
    o_ref[...] = (
        jnp.dot(h_ref[...], w_ref[...], preferred_element_type=jnp.float32) * dis
    )


def _mm1(hp, W, dA, dB):
    return pl.pallas_call(
        _mm1_body,
        grid=(GRID,),
        in_specs=[
            pl.BlockSpec((RB, DI), lambda i: (i, 0)),
            pl.BlockSpec((DI, DH), lambda i: (0, 0)),
            pl.BlockSpec((RB, 1), lambda i: (i, 0)),
            pl.BlockSpec((RB, 1), lambda i: (i, 0)),
        ],
        out_specs=pl.BlockSpec((RB, DH), lambda i: (i, 0)),
        out_shape=jax.ShapeDtypeStruct((NP, DH), jnp.float32),
    )(hp, W, dA, dB)


def _mm2_body(acc_ref, xt_ref, da_ref, db_ref, bg_ref, wf_ref, bf_ref, o_ref):
    dis = lax.rsqrt(da_ref[...] + db_ref[...] + 1.0)
    a = (acc_ref[...] + xt_ref[...]) * dis + bg_ref[...]
    z = jnp.maximum(a, 0.0)
    o_ref[...] = (
        jnp.dot(z, wf_ref[...], preferred_element_type=jnp.float32) + bf_ref[...]
    )


def _mm2(acc, xt, dA, dB, bg, Wf, bf):
    return pl.pallas_call(
        _mm2_body,
        grid=(GRID,),
        in_specs=[
            pl.BlockSpec((RB, DH), lambda i: (i, 0)),
            pl.BlockSpec((RB, DH), lambda i: (i, 0)),
            pl.BlockSpec((RB, 1), lambda i: (i, 0)),
            pl.BlockSpec((RB, 1), lambda i: (i, 0)),
            pl.BlockSpec((1, DH), lambda i: (0, 0)),
            pl.BlockSpec((DH, DO), lambda i: (0, 0)),
            pl.BlockSpec((1, DO), lambda i: (0, 0)),
        ],
        out_specs=pl.BlockSpec((RB, DO), lambda i: (i, 0)),
        out_shape=jax.ShapeDtypeStruct((NP, DO), jnp.float32),
    )(acc, xt, dA, dB, bg, Wf, bf)


def kernel(h, edge_index, W_gcn, b_gcn, W_fc, b_fc):
    h = h.astype(jnp.float32)
    ei = edge_index.astype(jnp.int32)
    src = jnp.pad(ei[0], (0, EPAD - E))
    dst = jnp.pad(ei[1], (0, EPAD - E), constant_values=TRASH)
    dst3d = dst.reshape(NW, NSTEP_D, CH)
    src3 = src.reshape(NS, NSTEP, CH)
    dst3 = dst.reshape(NS, NSTEP, CH)
    hp = jnp.pad(h, ((0, NP - N), (0, 0)))
    za = jnp.zeros((RPT,), jnp.float32)
    zb = jnp.zeros((CH, DH), jnp.float32)
    ones = jnp.ones((CH,), jnp.float32)

    deg = _deg_kernel(dst3d, za, ones)
    dA = deg[0].reshape(NP, 1)
    dB = deg[1].reshape(NP, 1)
    xt = _mm1(hp, W_gcn, dA, dB)
    acc = _scat_kernel(xt, src3, dst3, zb)
    out = _mm2(acc, xt, dA, dB, b_gcn.reshape(1, DH), W_fc, b_fc.reshape(1, DO))
    return out[:N]


# speedup vs baseline: 4.4520x; 1.2819x over previous
import jax
import jax.numpy as jnp
from jax import lax
from jax.experimental import pallas as pl
from jax.experimental.pallas import tpu as pltpu
from jax.experimental.pallas import tpu_sc as plsc

N = 10000
E = 320000
DI = 128
DH = 128
DO = 64

NC, NS = 2, 16
NW = NC * NS
NP = 10240
RPT = NP // NS
CH = 128
NSTEP_D = 80
NSTEP = 160
NBUF = 4
EPAD = NW * NSTEP_D * CH
TRASH = NP - 1

RB = 1024
GRID = NP // RB


def _deg_body(dst_hbm, zeros_hbm, ones_hbm, out_hbm, idx_v, ones_v, deg_sh, sem):
    cid = lax.axis_index("c")
    sid = lax.axis_index("s")
    wid = cid * NS + sid
    pltpu.sync_copy(dst_hbm.at[wid], idx_v)
    pltpu.sync_copy(ones_hbm, ones_v)
    pltpu.sync_copy(zeros_hbm, deg_sh.at[pl.ds(sid * RPT, RPT)])
    plsc.subcore_barrier()

    def fire(j, carry):
        pltpu.async_copy(ones_v, deg_sh.at[idx_v.at[j]], sem, add=True)
        return carry

    lax.fori_loop(0, NSTEP_D, fire, 0)

    def drain(j, carry):
        pltpu.make_async_copy(ones_v, deg_sh.at[idx_v.at[0]], sem).wait()
        return carry

    lax.fori_loop(0, NSTEP_D, drain, 0)
    plsc.subcore_barrier()
    pltpu.sync_copy(
        deg_sh.at[pl.ds(sid * RPT, RPT)], out_hbm.at[cid, pl.ds(sid * RPT, RPT)]
    )


def _deg_kernel(dst3d, za, ones):
    return pl.kernel(
        _deg_body,
        out_type=jax.ShapeDtypeStruct((NC, NP), jnp.float32),
        mesh=plsc.VectorSubcoreMesh(
            core_axis_name="c", subcore_axis_name="s", num_cores=NC, num_subcores=NS
        ),
        scratch_types=[
            pltpu.VMEM((NSTEP_D, CH), jnp.int32),
            pltpu.VMEM((CH,), jnp.float32),
            pltpu.VMEM_SHARED((NP,), jnp.float32),
            pltpu.SemaphoreType.DMA,
        ],
    )(dst3d, za, ones)


PASS_ROWS = 4096
NPASS = 3
TR = 128
ACCR = PASS_ROWS + TR
ZR = ACCR // NS
CPT = PASS_ROWS // NS
LASTR = NP - 2 * PASS_ROWS

PACK = 16384
PADV = PACK - 1


def _scat_body(xt_hbm, pk_hbm, zeros_hbm, out_hbm,
               pk2d, sstage, dstage, r0, r1, r2, r3,
               acc_sh, sg0, sg1, sg2, sg3, ss0, ss1, ss2, ss3):
    rows = (r0, r1, r2, r3)
    semg = (sg0, sg1, sg2, sg3)
    sems = (ss0, ss1, ss2, ss3)
    sid = lax.axis_index("s")
    pltpu.sync_copy(pk_hbm.at[sid], pk2d)

    for p in range(NPASS):
        base = p * PASS_ROWS
        pltpu.sync_copy(zeros_hbm, acc_sh.at[pl.ds(sid * ZR, CH)])
        pltpu.sync_copy(zeros_hbm, acc_sh.at[pl.ds(sid * ZR + CH, CH)])
        pltpu.sync_copy(
            zeros_hbm.at[pl.ds(0, ZR - 2 * CH)],
            acc_sh.at[pl.ds(sid * ZR + 2 * CH, ZR - 2 * CH)],
        )
        plsc.subcore_barrier()

        def unpack(c, u, base=base):
            for q in range(CH // 16):
                pkd = pk2d[c, pl.ds(q * 16, 16)]
                src = lax.shift_right_logical(pkd, 14)
                d = jnp.bitwise_and(pkd, PADV)
                l = d - base
                ok = jnp.logical_and(l >= 0, l < PASS_ROWS)
                t = PASS_ROWS + jnp.bitwise_and(d, TR - 1)
                sstage[u, pl.ds(q * 16, 16)] = src
                dstage[u, pl.ds(q * 16, 16)] = jnp.where(ok, l, t)

        def ring(g, carry):
            for u in range(NBUF):
                unpack(g * NBUF + u, u)
                pltpu.async_copy(xt_hbm.at[sstage.at[u]], rows[u], semg[u])
            for u in range(NBUF):
                pltpu.make_async_copy(
                    xt_hbm.at[sstage.at[u]], rows[u], semg[u]
                ).wait()
                pltpu.async_copy(
                    rows[u], acc_sh.at[dstage.at[u]], sems[u], add=True
                )
            for u in range(NBUF):
                pltpu.make_async_copy(
                    rows[u], acc_sh.at[dstage.at[u]], sems[u]
                ).wait()
            return carry

        lax.fori_loop(0, NSTEP // NBUF, ring, 0)
        plsc.subcore_barrier()

        if p < NPASS - 1:
            pltpu.sync_copy(
                acc_sh.at[pl.ds(sid * CPT, CPT)],
                out_hbm.at[pl.ds(base + sid * CPT, CPT)],
            )
        else:
            pltpu.sync_copy(
                acc_sh.at[pl.ds(sid * (LASTR // NS), LASTR // NS)],
                out_hbm.at[pl.ds(base + sid * (LASTR // NS), LASTR // NS)],
            )
        plsc.subcore_barrier()


def _scat_kernel(xt, pk3, zb):
    return pl.kernel(
        _scat_body,
        out_type=jax.ShapeDtypeStruct((NP, DH), jnp.float32),
        mesh=plsc.VectorSubcoreMesh(
            core_axis_name="c", subcore_axis_name="s", num_cores=1, num_subcores=NS
        ),
        scratch_types=[
            pltpu.VMEM((NSTEP, CH), jnp.int32),
            pltpu.VMEM((NBUF, CH), jnp.int32),
            pltpu.VMEM((NBUF, CH), jnp.int32),
            pltpu.VMEM((CH, DH), jnp.float32),
            pltpu.VMEM((CH, DH), jnp.float32),
            pltpu.VMEM((CH, DH), jnp.float32),
            pltpu.VMEM((CH, DH), jnp.float32),
            pltpu.VMEM_SHARED((ACCR, DH), jnp.float32),
            pltpu.SemaphoreType.DMA,
            pltpu.SemaphoreType.DMA,
            pltpu.SemaphoreType.DMA,
            pltpu.SemaphoreType.DMA,
            pltpu.SemaphoreType.DMA,
            pltpu.SemaphoreType.DMA,
            pltpu.SemaphoreType.DMA,
            pltpu.SemaphoreType.DMA,
        ],
    )(xt, pk3, zb)


def _mm1_body(h_ref, w_ref, da_ref, db_ref, o_ref):
    dis = lax.rsqrt(da_ref[...] + db_ref[...] + 1.0)
    o_ref[...] = (
        jnp.dot(h_ref[...], w_ref[...], preferred_element_type=jnp.float32) * dis
    )


def _mm1(hp, W, dA, dB):
    return pl.pallas_call(
        _mm1_body,
        grid=(GRID,),
        in_specs=[
            pl.BlockSpec((RB, DI), lambda i: (i, 0)),
            pl.BlockSpec((DI, DH), lambda i: (0, 0)),
            pl.BlockSpec((RB, 1), lambda i: (i, 0)),
            pl.BlockSpec((RB, 1), lambda i: (i, 0)),
        ],
        out_specs=pl.BlockSpec((RB, DH), lambda i: (i, 0)),
        out_shape=jax.ShapeDtypeStruct((NP, DH), jnp.float32),
    )(hp, W, dA, dB)


def _mm2_body(acc_ref, xt_ref, da_ref, db_ref, bg_ref, wf_ref, bf_ref, o_ref):
    dis = lax.rsqrt(da_ref[...] + db_ref[...] + 1.0)
    a = (acc_ref[...] + xt_ref[...]) * dis + bg_ref[...]
    z = jnp.maximum(a, 0.0)
    o_ref[...] = (
        jnp.dot(z, wf_ref[...], preferred_element_type=jnp.float32) + bf_ref[...]
    )


def _mm2(acc, xt, dA, dB, bg, Wf, bf):
    return pl.pallas_call(
        _mm2_body,
        grid=(GRID,),
        in_specs=[
            pl.BlockSpec((RB, DH), lambda i: (i, 0)),
            pl.BlockSpec((RB, DH), lambda i: (i, 0)),
            pl.BlockSpec((RB, 1), lambda i: (i, 0)),
            pl.BlockSpec((RB, 1), lambda i: (i, 0)),
            pl.BlockSpec((1, DH), lambda i: (0, 0)),
            pl.BlockSpec((DH, DO), lambda i: (0, 0)),
            pl.BlockSpec((1, DO), lambda i: (0, 0)),
        ],
        out_specs=pl.BlockSpec((RB, DO), lambda i: (i, 0)),
        out_shape=jax.ShapeDtypeStruct((NP, DO), jnp.float32),
    )(acc, xt, dA, dB, bg, Wf, bf)


def kernel(h, edge_index, W_gcn, b_gcn, W_fc, b_fc):
    h = h.astype(jnp.float32)
    ei = edge_index.astype(jnp.int32)
    src = jnp.pad(ei[0], (0, EPAD - E))
    dst = jnp.pad(ei[1], (0, EPAD - E), constant_values=TRASH)
    dst3d = dst.reshape(NW, NSTEP_D, CH)
    pk3 = (src * PACK + jnp.pad(ei[1], (0, EPAD - E), constant_values=PADV)
           ).reshape(NS, NSTEP, CH)
    hp = jnp.pad(h, ((0, NP - N), (0, 0)))
    za = jnp.zeros((RPT,), jnp.float32)
    zb = jnp.zeros((CH, DH), jnp.float32)
    ones = jnp.ones((CH,), jnp.float32)

    deg = _deg_kernel(dst3d, za, ones)
    dA = deg[0].reshape(NP, 1)
    dB = deg[1].reshape(NP, 1)
    xt = _mm1(hp, W_gcn, dA, dB)
    acc = _scat_kernel(xt, pk3, zb)
    out = _mm2(acc, xt, dA, dB, b_gcn.reshape(1, DH), W_fc, b_fc.reshape(1, DO))
    return out[:N]


# speedup vs baseline: 6.4728x; 1.4539x over previous
import jax
import jax.numpy as jnp
from jax import lax
from jax.experimental import pallas as pl
from jax.experimental.pallas import tpu as pltpu
from jax.experimental.pallas import tpu_sc as plsc

N = 10000
E = 320000
DI = 128
DH = 128
DO = 64

NC, NS = 2, 16
NP = 10240
CH = 128
NSTEP = 160
NBUF = 4
EPAD = NS * NSTEP * CH
ER = EPAD // CH

PASS_ROWS = 4096
NPASS = 3
TR = 128
ACCR = PASS_ROWS + TR
ZR = ACCR // NS
CPT = PASS_ROWS // NS
LASTR = NP - 2 * PASS_ROWS

PACK = 16384
PADV = PACK - 1
SROWS = 168
SFILL = SROWS * 2048 // NS

RB = 1024
GRID = NP // RB


def _perm_body(d_ref, perm_ref, meta_ref):
    d = d_ref[...]
    ik = lax.broadcasted_iota(jnp.int32, (CH, CH), 0)
    il = lax.broadcasted_iota(jnp.int32, (CH, CH), 1)
    t_inc = (ik <= il).astype(jnp.float32)
    t_exc = (il < ik).astype(jnp.float32)
    perm = jnp.zeros((ER, CH), jnp.float32)
    cb = jnp.int32(0)
    cbs = []
    for b in range(NPASS + 1):
        lo, hi = b * PASS_ROWS, (b + 1) * PASS_ROWS
        if b < NPASS:
            m = jnp.logical_and(d >= lo, d < hi).astype(jnp.float32)
        else:
            m = (d >= lo).astype(jnp.float32)
        lanecum = jnp.dot(m, t_inc, preferred_element_type=jnp.float32)
        rowtot = jnp.sum(m, axis=1, keepdims=True)
        prefs = []
        carry = jnp.zeros((1, 1), jnp.float32)
        for g in range(ER // CH):
            blk = rowtot[g * CH:(g + 1) * CH]
            pref = jnp.dot(t_exc, blk, preferred_element_type=jnp.float32)
            prefs.append(pref + carry)
            carry = carry + jnp.sum(blk, axis=0, keepdims=True)
        rowpref = jnp.concatenate(prefs, axis=0)
        rank = lanecum + rowpref
        perm = perm + m * (jnp.float32(128.0) * cb.astype(jnp.float32)
                           + rank - 1.0)
        cbs.append(cb)
        cnt = jnp.sum(m).astype(jnp.int32)
        cb = cb + lax.div(cnt + CH - 1, jnp.int32(CH))
    cbs.append(cb)
    perm_ref[...] = perm.astype(jnp.int32)
    lane = lax.broadcasted_iota(jnp.int32, (8, CH), 1)
    meta = jnp.zeros((8, CH), jnp.int32)
    for b in range(NPASS + 1):
        meta = meta + jnp.where(lane == b, cbs[b], 0)
    meta_ref[...] = meta


def _perm_kernel(d2):
    return pl.pallas_call(
        _perm_body,
        grid=(1,),
        in_specs=[pl.BlockSpec((ER, CH), lambda i: (0, 0))],
        out_specs=[
            pl.BlockSpec((ER, CH), lambda i: (0, 0)),
            pl.BlockSpec((8, CH), lambda i: (0, 0)),
        ],
        out_shape=[
            jax.ShapeDtypeStruct((ER, CH), jnp.int32),
            jax.ShapeDtypeStruct((8, CH), jnp.int32),
        ],
    )(d2)


def _deg_body(pk_hbm, perm_hbm, za_hbm, ones_hbm, fill_hbm,
              deg_hbm, srt_hbm, pk2d, perm2d, dstage, ones_v, deg_sh,
              sem, semp):
    sid = lax.axis_index("s")
    pltpu.sync_copy(pk_hbm.at[sid], pk2d)
    pltpu.sync_copy(perm_hbm.at[sid], perm2d)
    pltpu.sync_copy(ones_hbm, ones_v)
    pltpu.sync_copy(za_hbm, deg_sh.at[pl.ds(sid * (PACK // NS), PACK // NS)])
    pltpu.sync_copy(fill_hbm, srt_hbm.at[pl.ds(sid * SFILL, SFILL)])
    plsc.subcore_barrier()

    def body(c, carry):
        for q in range(CH // 16):
            pkd = pk2d[c, pl.ds(q * 16, 16)]
            dstage[c, pl.ds(q * 16, 16)] = jnp.bitwise_and(pkd, PADV)
        pltpu.async_copy(ones_v, deg_sh.at[dstage.at[c]], sem, add=True)
        pltpu.async_copy(pk2d.at[c], srt_hbm.at[perm2d.at[c]], semp)
        return carry

    lax.fori_loop(0, NSTEP, body, 0)

    def drain(c, carry):
        pltpu.make_async_copy(ones_v, deg_sh.at[dstage.at[0]], sem).wait()
        pltpu.make_async_copy(pk2d.at[0], srt_hbm.at[perm2d.at[0]], semp).wait()
        return carry

    lax.fori_loop(0, NSTEP, drain, 0)
    plsc.subcore_barrier()
    pltpu.sync_copy(
        deg_sh.at[pl.ds(sid * (NP // NS), NP // NS)],
        deg_hbm.at[pl.ds(sid * (NP // NS), NP // NS)],
    )


def _deg_kernel(pk3, perm3, za, ones, fill):
    return pl.kernel(
        _deg_body,
        out_type=(
            jax.ShapeDtypeStruct((NP,), jnp.float32),
            jax.ShapeDtypeStruct((SROWS * 2048,), jnp.int32),
        ),
        mesh=plsc.VectorSubcoreMesh(
            core_axis_name="c", subcore_axis_name="s", num_cores=1, num_subcores=NS
        ),
        scratch_types=[
            pltpu.VMEM((NSTEP, CH), jnp.int32),
            pltpu.VMEM((NSTEP, CH), jnp.int32),
            pltpu.VMEM((NSTEP, CH), jnp.int32),
            pltpu.VMEM((CH,), jnp.float32),
            pltpu.VMEM_SHARED((PACK,), jnp.float32),
            pltpu.SemaphoreType.DMA,
            pltpu.SemaphoreType.DMA,
        ],
    )(pk3, perm3, za, ones, fill)


def _scat_body(xt_hbm, srt_hbm, meta_hbm, zeros_hbm, out_hbm,
               pkloc, metav, sstage, dstage, r0, r1, r2, r3,
               acc_sh, sg0, sg1, sg2, sg3, ss0, ss1, ss2, ss3):
    rows = (r0, r1, r2, r3)
    semg = (sg0, sg1, sg2, sg3)
    sems = (ss0, ss1, ss2, ss3)
    sid = lax.axis_index("s")
    pltpu.sync_copy(srt_hbm.at[:, pl.ds(sid * CH, CH)], pkloc)
    pltpu.sync_copy(meta_hbm, metav)
    mvec = metav[pl.ds(0, 16)]

    for p in range(NPASS):
        base = p * PASS_ROWS
        cp0 = mvec[p]
        cp1 = mvec[p + 1]
        pltpu.sync_copy(zeros_hbm, acc_sh.at[pl.ds(sid * ZR, CH)])
        pltpu.sync_copy(zeros_hbm, acc_sh.at[pl.ds(sid * ZR + CH, CH)])
        pltpu.sync_copy(
            zeros_hbm.at[pl.ds(0, ZR - 2 * CH)],
            acc_sh.at[pl.ds(sid * ZR + 2 * CH, ZR - 2 * CH)],
        )
        plsc.subcore_barrier()

        j_lo = lax.div(cp0 - sid + NS - 1, jnp.int32(NS))
        j_hi = lax.div(cp1 - sid + NS - 1, jnp.int32(NS))
        ng = lax.div(j_hi - j_lo + NBUF - 1, jnp.int32(NBUF))

        def unpack(j, u, base=base):
            for q in range(CH // 16):
                pkd = pkloc[j, pl.ds(q * 16, 16)]
                src = lax.shift_right_logical(pkd, 14)
                d = jnp.bitwise_and(pkd, PADV)
                l = d - base
                ok = jnp.logical_and(l >= 0, l < PASS_ROWS)
                t = PASS_ROWS + jnp.bitwise_and(d, TR - 1)
                sstage[u, pl.ds(q * 16, 16)] = src
                dstage[u, pl.ds(q * 16, 16)] = jnp.where(ok, l, t)

        def ring(g, carry, j_lo=j_lo, j_hi=j_hi):
            for u in range(NBUF):
                j = j_lo + g * NBUF + u

                @pl.when(j < j_hi)
                def _(j=j, u=u):
                    unpack(j, u)
                    pltpu.async_copy(xt_hbm.at[sstage.at[u]], rows[u], semg[u])

            for u in range(NBUF):
                j = j_lo + g * NBUF + u

                @pl.when(j < j_hi)
                def _(u=u):
                    pltpu.make_async_copy(
                        xt_hbm.at[sstage.at[u]], rows[u], semg[u]
                    ).wait()
                    pltpu.async_copy(
                        rows[u], acc_sh.at[dstage.at[u]], sems[u], add=True
                    )

            for u in range(NBUF):
                j = j_lo + g * NBUF + u

                @pl.when(j < j_hi)
                def _(u=u):
                    pltpu.make_async_copy(
                        rows[u], acc_sh.at[dstage.at[u]], sems[u]
                    ).wait()

            return carry

        lax.fori_loop(0, ng, ring, 0)
        plsc.subcore_barrier()

        if p < NPASS - 1:
            pltpu.sync_copy(
                acc_sh.at[pl.ds(sid * CPT, CPT)],
                out_hbm.at[pl.ds(base + sid * CPT, CPT)],
            )
        else:
            pltpu.sync_copy(
                acc_sh.at[pl.ds(sid * (LASTR // NS), LASTR // NS)],
                out_hbm.at[pl.ds(base + sid * (LASTR // NS), LASTR // NS)],
            )
        plsc.subcore_barrier()


def _scat_kernel(xt, srt2, meta, zb):
    return pl.kernel(
        _scat_body,
        out_type=jax.ShapeDtypeStruct((NP, DH), jnp.float32),
        mesh=plsc.VectorSubcoreMesh(
            core_axis_name="c", subcore_axis_name="s", num_cores=1, num_subcores=NS
        ),
        scratch_types=[
            pltpu.VMEM((SROWS, CH), jnp.int32),
            pltpu.VMEM((CH,), jnp.int32),
            pltpu.VMEM((NBUF, CH), jnp.int32),
            pltpu.VMEM((NBUF, CH), jnp.int32),
            pltpu.VMEM((CH, DH), jnp.float32),
            pltpu.VMEM((CH, DH), jnp.float32),
            pltpu.VMEM((CH, DH), jnp.float32),
            pltpu.VMEM((CH, DH), jnp.float32),
            pltpu.VMEM_SHARED((ACCR, DH), jnp.float32),
            pltpu.SemaphoreType.DMA,
            pltpu.SemaphoreType.DMA,
            pltpu.SemaphoreType.DMA,
            pltpu.SemaphoreType.DMA,
            pltpu.SemaphoreType.DMA,
            pltpu.SemaphoreType.DMA,
            pltpu.SemaphoreType.DMA,
            pltpu.SemaphoreType.DMA,
        ],
    )(xt, srt2, meta, zb)


def _mm1_body(h_ref, w_ref, da_ref, o_ref):
    dis = lax.rsqrt(da_ref[...] + 1.0)
    o_ref[...] = (
        jnp.dot(h_ref[...], w_ref[...], preferred_element_type=jnp.float32) * dis
    )


def _mm1(hp, W, dA):
    return pl.pallas_call(
        _mm1_body,
        grid=(GRID,),
        in_specs=[
            pl.BlockSpec((RB, DI), lambda i: (i, 0)),
            pl.BlockSpec((DI, DH), lambda i: (0, 0)),
            pl.BlockSpec((RB, 1), lambda i: (i, 0)),
        ],
        out_specs=pl.BlockSpec((RB, DH), lambda i: (i, 0)),
        out_shape=jax.ShapeDtypeStruct((NP, DH), jnp.float32),
    )(hp, W, dA)


def _mm2_body(acc_ref, xt_ref, da_ref, bg_ref, wf_ref, bf_ref, o_ref):
    dis = lax.rsqrt(da_ref[...] + 1.0)
    a = (acc_ref[...] + xt_ref[...]) * dis + bg_ref[...]
    z = jnp.maximum(a, 0.0)
    o_ref[...] = (
        jnp.dot(z, wf_ref[...], preferred_element_type=jnp.float32) + bf_ref[...]
    )


def _mm2(acc, xt, dA, bg, Wf, bf):
    return pl.pallas_call(
        _mm2_body,
        grid=(GRID,),
        in_specs=[
            pl.BlockSpec((RB, DH), lambda i: (i, 0)),
            pl.BlockSpec((RB, DH), lambda i: (i, 0)),
            pl.BlockSpec((RB, 1), lambda i: (i, 0)),
            pl.BlockSpec((1, DH), lambda i: (0, 0)),
            pl.BlockSpec((DH, DO), lambda i: (0, 0)),
            pl.BlockSpec((1, DO), lambda i: (0, 0)),
        ],
        out_specs=pl.BlockSpec((RB, DO), lambda i: (i, 0)),
        out_shape=jax.ShapeDtypeStruct((NP, DO), jnp.float32),
    )(acc, xt, dA, bg, Wf, bf)


def kernel(h, edge_index, W_gcn, b_gcn, W_fc, b_fc):
    h = h.astype(jnp.float32)
    ei = edge_index.astype(jnp.int32)
    src = jnp.pad(ei[0], (0, EPAD - E))
    dst = jnp.pad(ei[1], (0, EPAD - E), constant_values=PADV)
    pk = src * PACK + dst
    pk3 = pk.reshape(NS, NSTEP, CH)
    d2 = dst.reshape(ER, CH)
    hp = jnp.pad(h, ((0, NP - N), (0, 0)))
    za = jnp.zeros((PACK // NS,), jnp.float32)
    zb = jnp.zeros((CH, DH), jnp.float32)
    ones = jnp.ones((CH,), jnp.float32)
    fill = jnp.full((SFILL,), PADV, jnp.int32)

    perm, meta = _perm_kernel(d2)
    perm3 = perm.reshape(NS, NSTEP, CH)
    deg, srt = _deg_kernel(pk3, perm3, za, ones, fill)
    dA = deg.reshape(NP, 1)
    xt = _mm1(hp, W_gcn, dA)
    srt2 = srt.reshape(SROWS, 2048)
    acc = _scat_kernel(xt, srt2, meta[0], zb)
    out = _mm2(acc, xt, dA, b_gcn.reshape(1, DH), W_fc, b_fc.reshape(1, DO))
    return out[:N]


# speedup vs baseline: 20.6837x; 3.1955x over previous
import jax
import jax.numpy as jnp
from jax import lax
from jax.experimental import pallas as pl
from jax.experimental.pallas import tpu as pltpu
from jax.experimental.pallas import tpu_sc as plsc

N = 10000
E = 320000
DI = 128
DH = 128
DO = 64

NC, NS = 2, 16
NP = 10240
CH = 128
NSTEP = 160
NBUF = 4
EPAD = NS * NSTEP * CH
ER = EPAD // CH

PASS_ROWS = 4096
NPASS = 3
TR = 128
ACCR = PASS_ROWS + TR
ZR = ACCR // NS
CPT = PASS_ROWS // NS
LASTR = NP - 2 * PASS_ROWS

PACK = 16384
PADV = PACK - 1
SROWS = 168
SFILL = SROWS * 2048 // NS

RB = 1024
GRID = NP // RB


def _perm_body(d_ref, perm_ref, meta_ref):
    d = d_ref[...]
    ik = lax.broadcasted_iota(jnp.int32, (CH, CH), 0)
    il = lax.broadcasted_iota(jnp.int32, (CH, CH), 1)
    t_inc = (ik <= il).astype(jnp.float32)
    t_exc = (il < ik).astype(jnp.float32)
    perm = jnp.zeros((ER, CH), jnp.float32)
    cb = jnp.int32(0)
    cbs = []
    for b in range(NPASS + 1):
        lo, hi = b * PASS_ROWS, (b + 1) * PASS_ROWS
        if b < NPASS:
            m = jnp.logical_and(d >= lo, d < hi).astype(jnp.float32)
        else:
            m = (d >= lo).astype(jnp.float32)
        lanecum = jnp.dot(m, t_inc, preferred_element_type=jnp.float32)
        rowtot = jnp.sum(m, axis=1, keepdims=True)
        prefs = []
        carry = jnp.zeros((1, 1), jnp.float32)
        for g in range(ER // CH):
            blk = rowtot[g * CH:(g + 1) * CH]
            pref = jnp.dot(t_exc, blk, preferred_element_type=jnp.float32)
            prefs.append(pref + carry)
            carry = carry + jnp.sum(blk, axis=0, keepdims=True)
        rowpref = jnp.concatenate(prefs, axis=0)
        rank = lanecum + rowpref
        perm = perm + m * (jnp.float32(128.0) * cb.astype(jnp.float32)
                           + rank - 1.0)
        cbs.append(cb)
        cnt = jnp.sum(m).astype(jnp.int32)
        cb = cb + lax.div(cnt + CH - 1, jnp.int32(CH))
    cbs.append(cb)
    perm_ref[...] = perm.astype(jnp.int32)
    lane = lax.broadcasted_iota(jnp.int32, (8, CH), 1)
    meta = jnp.zeros((8, CH), jnp.int32)
    for b in range(NPASS + 1):
        meta = meta + jnp.where(lane == b, cbs[b], 0)
    meta_ref[...] = meta


def _perm_kernel(d2):
    return pl.pallas_call(
        _perm_body,
        grid=(1,),
        in_specs=[pl.BlockSpec((ER, CH), lambda i: (0, 0))],
        out_specs=[
            pl.BlockSpec((ER, CH), lambda i: (0, 0)),
            pl.BlockSpec((8, CH), lambda i: (0, 0)),
        ],
        out_shape=[
            jax.ShapeDtypeStruct((ER, CH), jnp.int32),
            jax.ShapeDtypeStruct((8, CH), jnp.int32),
        ],
    )(d2)


def _deg_body(pk_hbm, perm_hbm, za_hbm, ones_hbm, fill_hbm,
              deg_hbm, srt_hbm, pk2d, perm2d, dstage, ones_v, deg_sh, srt_sh,
              sem, semp):
    sid = lax.axis_index("s")
    pltpu.sync_copy(pk_hbm.at[sid], pk2d)
    pltpu.sync_copy(perm_hbm.at[sid], perm2d)
    pltpu.sync_copy(ones_hbm, ones_v)
    pltpu.sync_copy(za_hbm, deg_sh.at[pl.ds(sid * (PACK // NS), PACK // NS)])
    pltpu.sync_copy(fill_hbm, srt_sh.at[pl.ds(sid * SFILL, SFILL)])
    plsc.subcore_barrier()

    def body(c, carry):
        for q in range(CH // 16):
            pkd = pk2d[c, pl.ds(q * 16, 16)]
            dstage[c, pl.ds(q * 16, 16)] = jnp.bitwise_and(pkd, PADV)
        pltpu.async_copy(ones_v, deg_sh.at[dstage.at[c]], sem, add=True)
        pltpu.async_copy(pk2d.at[c], srt_sh.at[perm2d.at[c]], semp)
        return carry

    lax.fori_loop(0, NSTEP, body, 0)

    def drain(c, carry):
        pltpu.make_async_copy(ones_v, deg_sh.at[dstage.at[0]], sem).wait()
        pltpu.make_async_copy(pk2d.at[0], srt_sh.at[perm2d.at[0]], semp).wait()
        return carry

    lax.fori_loop(0, NSTEP, drain, 0)
    plsc.subcore_barrier()
    pltpu.sync_copy(
        deg_sh.at[pl.ds(sid * (NP // NS), NP // NS)],
        deg_hbm.at[pl.ds(sid * (NP // NS), NP // NS)],
    )
    pltpu.sync_copy(
        srt_sh.at[pl.ds(sid * SFILL, SFILL)],
        srt_hbm.at[pl.ds(sid * SFILL, SFILL)],
    )


def _deg_kernel(pk3, perm3, za, ones, fill):
    return pl.kernel(
        _deg_body,
        out_type=(
            jax.ShapeDtypeStruct((NP,), jnp.float32),
            jax.ShapeDtypeStruct((SROWS * 2048,), jnp.int32),
        ),
        mesh=plsc.VectorSubcoreMesh(
            core_axis_name="c", subcore_axis_name="s", num_cores=1, num_subcores=NS
        ),
        scratch_types=[
            pltpu.VMEM((NSTEP, CH), jnp.int32),
            pltpu.VMEM((NSTEP, CH), jnp.int32),
            pltpu.VMEM((NSTEP, CH), jnp.int32),
            pltpu.VMEM((CH,), jnp.float32),
            pltpu.VMEM_SHARED((PACK,), jnp.float32),
            pltpu.VMEM_SHARED((SROWS * 2048,), jnp.int32),
            pltpu.SemaphoreType.DMA,
            pltpu.SemaphoreType.DMA,
        ],
    )(pk3, perm3, za, ones, fill)


def _scat_body(xt_hbm, srt_hbm, meta_hbm, zeros_hbm, out_hbm,
               pkloc, metav, sstage, dstage, r0, r1, r2, r3,
               acc_sh, sg0, sg1, sg2, sg3, ss0, ss1, ss2, ss3):
    rows = (r0, r1, r2, r3)
    semg = (sg0, sg1, sg2, sg3)
    sems = (ss0, ss1, ss2, ss3)
    sid = lax.axis_index("s")
    pltpu.sync_copy(srt_hbm.at[:, pl.ds(sid * CH, CH)], pkloc)
    pltpu.sync_copy(meta_hbm, metav)
    mvec = metav[pl.ds(0, 16)]

    for p in range(NPASS):
        base = p * PASS_ROWS
        cp0 = mvec[p]
        cp1 = mvec[p + 1]
        pltpu.sync_copy(zeros_hbm, acc_sh.at[pl.ds(sid * ZR, CH)])
        pltpu.sync_copy(zeros_hbm, acc_sh.at[pl.ds(sid * ZR + CH, CH)])
        pltpu.sync_copy(
            zeros_hbm.at[pl.ds(0, ZR - 2 * CH)],
            acc_sh.at[pl.ds(sid * ZR + 2 * CH, ZR - 2 * CH)],
        )
        plsc.subcore_barrier()

        j_lo = lax.div(cp0 - sid + NS - 1, jnp.int32(NS))
        j_hi = lax.div(cp1 - sid + NS - 1, jnp.int32(NS))
        ng = lax.div(j_hi - j_lo + NBUF - 1, jnp.int32(NBUF))

        def unpack(j, u, base=base):
            for q in range(CH // 16):
                pkd = pkloc[j, pl.ds(q * 16, 16)]
                src = lax.shift_right_logical(pkd, 14)
                d = jnp.bitwise_and(pkd, PADV)
                l = d - base
                ok = jnp.logical_and(l >= 0, l < PASS_ROWS)
                t = PASS_ROWS + jnp.bitwise_and(d, TR - 1)
                sstage[u, pl.ds(q * 16, 16)] = src
                dstage[u, pl.ds(q * 16, 16)] = jnp.where(ok, l, t)

        def ring(g, carry, j_lo=j_lo, j_hi=j_hi):
            for u in range(NBUF):
                j = j_lo + g * NBUF + u

                @pl.when(j < j_hi)
                def _(j=j, u=u):
                    unpack(j, u)
                    pltpu.async_copy(xt_hbm.at[sstage.at[u]], rows[u], semg[u])

            for u in range(NBUF):
                j = j_lo + g * NBUF + u

                @pl.when(j < j_hi)
                def _(u=u):
                    pltpu.make_async_copy(
                        xt_hbm.at[sstage.at[u]], rows[u], semg[u]
                    ).wait()
                    pltpu.async_copy(
                        rows[u], acc_sh.at[dstage.at[u]], sems[u], add=True
                    )

            for u in range(NBUF):
                j = j_lo + g * NBUF + u

                @pl.when(j < j_hi)
                def _(u=u):
                    pltpu.make_async_copy(
                        rows[u], acc_sh.at[dstage.at[u]], sems[u]
                    ).wait()

            return carry

        lax.fori_loop(0, ng, ring, 0)
        plsc.subcore_barrier()

        if p < NPASS - 1:
            pltpu.sync_copy(
                acc_sh.at[pl.ds(sid * CPT, CPT)],
                out_hbm.at[pl.ds(base + sid * CPT, CPT)],
            )
        else:
            pltpu.sync_copy(
                acc_sh.at[pl.ds(sid * (LASTR // NS), LASTR // NS)],
                out_hbm.at[pl.ds(base + sid * (LASTR // NS), LASTR // NS)],
            )
        plsc.subcore_barrier()


def _scat_kernel(xt, srt2, meta, zb):
    return pl.kernel(
        _scat_body,
        out_type=jax.ShapeDtypeStruct((NP, DH), jnp.float32),
        mesh=plsc.VectorSubcoreMesh(
            core_axis_name="c", subcore_axis_name="s", num_cores=1, num_subcores=NS
        ),
        scratch_types=[
            pltpu.VMEM((SROWS, CH), jnp.int32),
            pltpu.VMEM((CH,), jnp.int32),
            pltpu.VMEM((NBUF, CH), jnp.int32),
            pltpu.VMEM((NBUF, CH), jnp.int32),
            pltpu.VMEM((CH, DH), jnp.float32),
            pltpu.VMEM((CH, DH), jnp.float32),
            pltpu.VMEM((CH, DH), jnp.float32),
            pltpu.VMEM((CH, DH), jnp.float32),
            pltpu.VMEM_SHARED((ACCR, DH), jnp.float32),
            pltpu.SemaphoreType.DMA,
            pltpu.SemaphoreType.DMA,
            pltpu.SemaphoreType.DMA,
            pltpu.SemaphoreType.DMA,
            pltpu.SemaphoreType.DMA,
            pltpu.SemaphoreType.DMA,
            pltpu.SemaphoreType.DMA,
            pltpu.SemaphoreType.DMA,
        ],
    )(xt, srt2, meta, zb)


def _mm1_body(h_ref, w_ref, da_ref, o_ref):
    dis = lax.rsqrt(da_ref[...] + 1.0)
    o_ref[...] = (
        jnp.dot(h_ref[...], w_ref[...], preferred_element_type=jnp.float32) * dis
    )


def _mm1(hp, W, dA):
    return pl.pallas_call(
        _mm1_body,
        grid=(GRID,),
        in_specs=[
            pl.BlockSpec((RB, DI), lambda i: (i, 0)),
            pl.BlockSpec((DI, DH), lambda i: (0, 0)),
            pl.BlockSpec((RB, 1), lambda i: (i, 0)),
        ],
        out_specs=pl.BlockSpec((RB, DH), lambda i: (i, 0)),
        out_shape=jax.ShapeDtypeStruct((NP, DH), jnp.float32),
    )(hp, W, dA)


def _mm2_body(acc_ref, xt_ref, da_ref, bg_ref, wf_ref, bf_ref, o_ref):
    dis = lax.rsqrt(da_ref[...] + 1.0)
    a = (acc_ref[...] + xt_ref[...]) * dis + bg_ref[...]
    z = jnp.maximum(a, 0.0)
    o_ref[...] = (
        jnp.dot(z, wf_ref[...], preferred_element_type=jnp.float32) + bf_ref[...]
    )


def _mm2(acc, xt, dA, bg, Wf, bf):
    return pl.pallas_call(
        _mm2_body,
        grid=(GRID,),
        in_specs=[
            pl.BlockSpec((RB, DH), lambda i: (i, 0)),
            pl.BlockSpec((RB, DH), lambda i: (i, 0)),
            pl.BlockSpec((RB, 1), lambda i: (i, 0)),
            pl.BlockSpec((1, DH), lambda i: (0, 0)),
            pl.BlockSpec((DH, DO), lambda i: (0, 0)),
            pl.BlockSpec((1, DO), lambda i: (0, 0)),
        ],
        out_specs=pl.BlockSpec((RB, DO), lambda i: (i, 0)),
        out_shape=jax.ShapeDtypeStruct((NP, DO), jnp.float32),
    )(acc, xt, dA, bg, Wf, bf)


def kernel(h, edge_index, W_gcn, b_gcn, W_fc, b_fc):
    h = h.astype(jnp.float32)
    ei = edge_index.astype(jnp.int32)
    src = jnp.pad(ei[0], (0, EPAD - E))
    dst = jnp.pad(ei[1], (0, EPAD - E), constant_values=PADV)
    pk = src * PACK + dst
    pk3 = pk.reshape(NS, NSTEP, CH)
    d2 = dst.reshape(ER, CH)
    hp = jnp.pad(h, ((0, NP - N), (0, 0)))
    za = jnp.zeros((PACK // NS,), jnp.float32)
    zb = jnp.zeros((CH, DH), jnp.float32)
    ones = jnp.ones((CH,), jnp.float32)
    fill = jnp.full((SFILL,), PADV, jnp.int32)

    perm, meta = _perm_kernel(d2)
    perm3 = perm.reshape(NS, NSTEP, CH)
    deg, srt = _deg_kernel(pk3, perm3, za, ones, fill)
    dA = deg.reshape(NP, 1)
    xt = _mm1(hp, W_gcn, dA)
    srt2 = srt.reshape(SROWS, 2048)
    acc = _scat_kernel(xt, srt2, meta[0], zb)
    out = _mm2(acc, xt, dA, b_gcn.reshape(1, DH), W_fc, b_fc.reshape(1, DO))
    return out[:N]
